# baseline (speedup 1.0000x reference)
import jax
import jax.numpy as jnp
import numpy as np
from jax import lax
from jax.experimental import pallas as pl
from jax.experimental.pallas import tpu as pltpu

_FACTOR = 3
_B, _L, _H, _D = 1, 2048, 12, 64
_SAMPLE_K = 2 * _FACTOR * int(np.ceil(np.log(_L)))
_NTOP = _FACTOR * int(np.ceil(np.log(_L)))
_SCALE = 1.0 / float(np.sqrt(_D))

_BQ = 1024
_NQB = _L // _BQ
_BR = 2048
_NRB = _L // _BR


def _threefry_raw(k1, k2, x1, x2):
    u32 = np.uint32
    def rotl(x, d):
        return (x << u32(d)) | (x >> u32(32 - d))
    ks = [u32(k1), u32(k2), u32(k1) ^ u32(k2) ^ u32(0x1BD11BDA)]
    rotations = [(13, 15, 26, 6), (17, 29, 16, 24)]
    x = [x1.astype(np.uint32) + ks[0], x2.astype(np.uint32) + ks[1]]
    for i in range(5):
        for r in rotations[i % 2]:
            x[0] = x[0] + x[1]
            x[1] = rotl(x[1], r)
            x[1] = x[0] ^ x[1]
        x[0] = x[0] + ks[(i + 1) % 3]
        x[1] = x[1] + ks[(i + 2) % 3] + u32(i + 1)
    return x[0], x[1]


def _build_tables():
    k1, k2 = np.uint32(0), np.uint32(42)
    b1, b2 = _threefry_raw(k1, k2, np.zeros(2, np.uint32),
                           np.arange(2, dtype=np.uint32))
    n = _L * _SAMPLE_K
    o1, o2 = _threefry_raw(b1[1], b2[1], np.zeros(n, np.uint32),
                           np.arange(n, dtype=np.uint32))
    idx = ((o1 ^ o2) % np.uint32(_L)).astype(np.int64).reshape(_L, _SAMPLE_K)
    count = np.zeros((_L, _L), dtype=np.int8)
    np.add.at(count, (np.arange(_L)[:, None], idx), 1)
    return count


_TABLE_CACHE: list = []


def _get_tables():
    if not _TABLE_CACHE:
        _TABLE_CACHE.append(_build_tables())
    return _TABLE_CACHE[0]


def _score_body(q_ref, k_ref, c_ref, m_ref):
    q = q_ref[0]
    k = k_ref[0]
    cnt = c_ref[...].astype(jnp.float32)
    s = lax.dot_general(q, k, (((1,), (1,)), ((), ())),
                        precision=lax.Precision.HIGHEST,
                        preferred_element_type=jnp.float32)
    mx = jnp.max(jnp.where(cnt > 0, s, -jnp.inf), axis=1)
    sm = jnp.sum(s * cnt, axis=1) / _L
    m_ref[0, 0, :] = mx - sm


def _select_body(m_ref, q_ref, k_ref, idx_ref, attn_ref, oh_ref):
    iota = lax.broadcasted_iota(jnp.int32, (1, _L), 1)

    def body(u, mcur):
        mxv = jnp.max(mcur)
        is_mx = mcur == mxv
        idx_u = jnp.min(jnp.where(is_mx, iota, _L))
        sel = iota == idx_u
        oh_ref[pl.ds(u, 1), :] = sel.astype(jnp.float32)
        return jnp.where(sel, -jnp.inf, mcur)

    lax.fori_loop(0, _NTOP, body, m_ref[0])

    oh = oh_ref[...]
    lane = lax.broadcasted_iota(jnp.int32, (_NTOP, _L), 1).astype(jnp.float32)
    idx_ref[0, 0, :] = jnp.sum(oh * lane, axis=1).astype(jnp.int32)

    qs = lax.dot_general(oh, q_ref[0], (((1,), (0,)), ((), ())),
                         precision=lax.Precision.HIGHEST,
                         preferred_element_type=jnp.float32)
    s = lax.dot_general(qs, k_ref[0], (((1,), (1,)), ((), ())),
                        precision=lax.Precision.HIGHEST,
                        preferred_element_type=jnp.float32)
    s = s * _SCALE
    s = s - jnp.max(s, axis=1, keepdims=True)
    e = jnp.exp(s)
    attn_ref[0] = e / jnp.sum(e, axis=1, keepdims=True)


def _write_body(idx_ref, attn_ref, o_ref):
    h = pl.program_id(0)
    rb = pl.program_id(1)
    base = rb * _BR
    o_ref[0] = jnp.zeros((_BR, _L), jnp.float32)
    for u in range(_NTOP):
        off = idx_ref[h, 0, u] - base

        @pl.when((off >= 0) & (off < _BR))
        def _copy(off=off, u=u):
            o_ref[0, pl.ds(off, 1), :] = attn_ref[0, pl.ds(u, 1), :]


@jax.jit
def kernel(queries, keys):
    q = jnp.transpose(queries[0], (1, 0, 2))
    k = jnp.transpose(keys[0], (1, 0, 2))
    cnt = jnp.asarray(_get_tables())

    m = pl.pallas_call(
        _score_body,
        grid=(_NQB, _H),
        in_specs=[
            pl.BlockSpec((1, _BQ, _D), lambda qb, h: (h, qb, 0)),
            pl.BlockSpec((1, _L, _D), lambda qb, h: (h, 0, 0)),
            pl.BlockSpec((_BQ, _L), lambda qb, h: (qb, 0)),
        ],
        out_specs=pl.BlockSpec((1, 1, _BQ), lambda qb, h: (h * _NQB + qb, 0, 0)),
        out_shape=jax.ShapeDtypeStruct((_H * _NQB, 1, _BQ), jnp.float32),
    )(q, k, cnt)

    m = m.reshape(_H, 1, _L)

    idx, attn = pl.pallas_call(
        _select_body,
        grid=(_H,),
        in_specs=[
            pl.BlockSpec((1, 1, _L), lambda h: (h, 0, 0)),
            pl.BlockSpec((1, _L, _D), lambda h: (h, 0, 0)),
            pl.BlockSpec((1, _L, _D), lambda h: (h, 0, 0)),
        ],
        out_specs=[
            pl.BlockSpec((1, 1, _NTOP), lambda h: (h, 0, 0)),
            pl.BlockSpec((1, _NTOP, _L), lambda h: (h, 0, 0)),
        ],
        out_shape=[
            jax.ShapeDtypeStruct((_H, 1, _NTOP), jnp.int32),
            jax.ShapeDtypeStruct((_H, _NTOP, _L), jnp.float32),
        ],
        scratch_shapes=[pltpu.VMEM((_NTOP, _L), jnp.float32)],
    )(m, q, k)

    out = pl.pallas_call(
        _write_body,
        grid=(_H, _NRB),
        in_specs=[
            pl.BlockSpec(memory_space=pltpu.SMEM),
            pl.BlockSpec((1, _NTOP, _L), lambda h, rb: (h, 0, 0)),
        ],
        out_specs=pl.BlockSpec((1, _BR, _L), lambda h, rb: (h, rb, 0)),
        out_shape=jax.ShapeDtypeStruct((_H, _L, _L), jnp.float32),
    )(idx, attn)

    return out.reshape(_B, _H, _L, _L)


# speedup vs baseline: 1.0305x; 1.0305x over previous
import jax
import jax.numpy as jnp
import numpy as np
from jax import lax
from jax.experimental import pallas as pl
from jax.experimental.pallas import tpu as pltpu

_FACTOR = 3
_B, _L, _H, _D = 1, 2048, 12, 64
_SAMPLE_K = 2 * _FACTOR * int(np.ceil(np.log(_L)))
_NTOP = _FACTOR * int(np.ceil(np.log(_L)))
_SCALE = 1.0 / float(np.sqrt(_D))

_BQ = 512
_NQC = _L // _BQ
_BR = 512
_NRB = _L // _BR


def _threefry_raw(k1, k2, x1, x2):
    u32 = np.uint32
    def rotl(x, d):
        return (x << u32(d)) | (x >> u32(32 - d))
    ks = [u32(k1), u32(k2), u32(k1) ^ u32(k2) ^ u32(0x1BD11BDA)]
    rotations = [(13, 15, 26, 6), (17, 29, 16, 24)]
    x = [x1.astype(np.uint32) + ks[0], x2.astype(np.uint32) + ks[1]]
    for i in range(5):
        for r in rotations[i % 2]:
            x[0] = x[0] + x[1]
            x[1] = rotl(x[1], r)
            x[1] = x[0] ^ x[1]
        x[0] = x[0] + ks[(i + 1) % 3]
        x[1] = x[1] + ks[(i + 2) % 3] + u32(i + 1)
    return x[0], x[1]


def _build_count():
    k1, k2 = np.uint32(0), np.uint32(42)
    b1, b2 = _threefry_raw(k1, k2, np.zeros(2, np.uint32),
                           np.arange(2, dtype=np.uint32))
    n = _L * _SAMPLE_K
    o1, o2 = _threefry_raw(b1[1], b2[1], np.zeros(n, np.uint32),
                           np.arange(n, dtype=np.uint32))
    idx = ((o1 ^ o2) % np.uint32(_L)).astype(np.int64).reshape(_L, _SAMPLE_K)
    count = np.zeros((_L, _L), dtype=np.int8)
    np.add.at(count, (np.arange(_L)[:, None], idx), 1)
    return count


_TABLE_CACHE: list = []


def _get_count():
    if not _TABLE_CACHE:
        _TABLE_CACHE.append(_build_count())
    return _TABLE_CACHE[0]


def _fused_body(q_ref, k_ref, c_ref, o_ref, attn_s, idx_s):
    rb = pl.program_id(1)

    @pl.when(rb == 0)
    def _compute():
        k = k_ref[0]

        m_parts = []
        for qc in range(_NQC):
            q = q_ref[0, qc * _BQ:(qc + 1) * _BQ, :]
            cnt = c_ref[qc * _BQ:(qc + 1) * _BQ, :].astype(jnp.float32)
            s = lax.dot_general(q, k, (((1,), (1,)), ((), ())),
                                precision=lax.Precision.HIGHEST,
                                preferred_element_type=jnp.float32)
            mx = jnp.max(jnp.where(cnt > 0, s, -jnp.inf), axis=1)
            sm = jnp.sum(s * cnt, axis=1) / _L
            m_parts.append((mx - sm).reshape(1, _BQ))
        mcur = jnp.concatenate(m_parts, axis=1)

        iota = lax.broadcasted_iota(jnp.int32, (1, _L), 1)
        oh_rows = []
        for u in range(_NTOP):
            mxv = jnp.max(mcur)
            idx_u = jnp.min(jnp.where(mcur == mxv, iota, _L))
            idx_s[u] = idx_u
            sel = iota == idx_u
            oh_rows.append(sel.astype(jnp.float32))
            mcur = jnp.where(sel, -jnp.inf, mcur)
        oh = jnp.concatenate(oh_rows, axis=0)

        qs = lax.dot_general(oh, q_ref[0], (((1,), (0,)), ((), ())),
                             precision=lax.Precision.HIGHEST,
                             preferred_element_type=jnp.float32)
        s = lax.dot_general(qs, k, (((1,), (1,)), ((), ())),
                            precision=lax.Precision.HIGHEST,
                            preferred_element_type=jnp.float32)
        s = s * _SCALE
        s = s - jnp.max(s, axis=1, keepdims=True)
        e = jnp.exp(s)
        attn_s[...] = e / jnp.sum(e, axis=1, keepdims=True)

    base = rb * _BR
    o_ref[0] = jnp.zeros((_BR, _L), jnp.float32)
    for u in range(_NTOP):
        off = idx_s[u] - base

        @pl.when((off >= 0) & (off < _BR))
        def _copy(off=off, u=u):
            o_ref[0, pl.ds(off, 1), :] = attn_s[pl.ds(u, 1), :]


@jax.jit
def kernel(queries, keys):
    q = jnp.transpose(queries[0], (1, 0, 2))
    k = jnp.transpose(keys[0], (1, 0, 2))
    cnt = jnp.asarray(_get_count())

    out = pl.pallas_call(
        _fused_body,
        grid=(_H, _NRB),
        in_specs=[
            pl.BlockSpec((1, _L, _D), lambda h, rb: (h, 0, 0)),
            pl.BlockSpec((1, _L, _D), lambda h, rb: (h, 0, 0)),
            pl.BlockSpec((_L, _L), lambda h, rb: (0, 0)),
        ],
        out_specs=pl.BlockSpec((1, _BR, _L), lambda h, rb: (h, rb, 0)),
        out_shape=jax.ShapeDtypeStruct((_H, _L, _L), jnp.float32),
        scratch_shapes=[
            pltpu.VMEM((_NTOP, _L), jnp.float32),
            pltpu.SMEM((_NTOP,), jnp.int32),
        ],
    )(q, k, cnt)

    return out.reshape(_B, _H, _L, _L)
